# baseline (speedup 1.0000x reference)
import jax
import jax.numpy as jnp
from jax.experimental import pallas as pl
from jax.experimental.pallas import tpu as pltpu

N = 512
RCUT = 6.0
RCUT_SMTH = 0.5
SR_MEAN = 0.1
SR_STD = 0.3
NNBRS = 128.0
AXIS = 16
WID1 = 32
WID2 = 64
FIT = 128
OUT_NORM = 1.0
EBIAS = 0.0

GRP = 8


def _dp_kernel(box_ref, bf3_ref, c_ref, W1_ref, b1_ref, W2_ref, b2_ref,
               tb_ref, Wf1_ref, bf1_ref, Wf2_ref, bf2_ref, Wf3_ref, out_ref,
               snc_ref, r0_ref, r1_ref, r2s_ref, r3_ref, w2p_ref,
               t0_ref, t1_ref, t2_ref, t3_ref):
    coord = c_ref[...]
    cT = jnp.transpose(coord)

    diffs = []
    r2 = jnp.full((N, N), 1e-16, jnp.float32)
    for d in range(3):
        Ld = box_ref[d, d]
        dd = coord[:, d:d + 1] - cT[d:d + 1, :]
        fr = dd * (1.0 / Ld)
        fr = fr - jnp.round(fr)
        dd = fr * Ld
        diffs.append(dd)
        r2 = r2 + dd * dd
    r = jnp.sqrt(r2)

    u = (r - RCUT_SMTH) / (RCUT - RCUT_SMTH)
    u = jnp.clip(u, 0.0, 1.0)
    sw = u * u * u * (-6.0 * u * u + 15.0 * u - 10.0) + 1.0
    inv_r = 1.0 / jnp.maximum(r, 1e-8)
    sr = jnp.where(r < RCUT, inv_r * sw, 0.0)
    rows = jax.lax.broadcasted_iota(jnp.int32, (N, N), 0)
    cols = jax.lax.broadcasted_iota(jnp.int32, (N, N), 1)
    sr = jnp.where(rows == cols, 0.0, sr)

    srn = sr / SR_STD
    snc_ref[...] = (sr - SR_MEAN) / SR_STD
    inv_rr = 1.0 / (r + 1e-16)
    sq3 = 3.0 ** 0.5
    r0_ref[...] = srn
    r1_ref[...] = sq3 * srn * (diffs[0] * inv_rr)
    r2s_ref[...] = sq3 * srn * (diffs[1] * inv_rr)
    r3_ref[...] = sq3 * srn * (diffs[2] * inv_rr)

    W2T = jnp.transpose(W2_ref[...])
    A = jnp.broadcast_to(W2T[None], (GRP, WID2, WID1)).reshape(GRP * WID2, WID1)
    ci = jax.lax.broadcasted_iota(jnp.int32, (WID1, WID1 * GRP), 0)
    cj = jax.lax.broadcasted_iota(jnp.int32, (WID1, WID1 * GRP), 1)
    Sel = jnp.where(ci == cj // GRP, 1.0, 0.0).astype(jnp.float32)
    W2e = jnp.dot(A, Sel, preferred_element_type=jnp.float32)
    pi = jax.lax.broadcasted_iota(jnp.int32, (GRP * WID2, WID1 * GRP), 0)
    qi = jax.lax.broadcasted_iota(jnp.int32, (GRP * WID2, WID1 * GRP), 1)
    w2p_ref[...] = jnp.where((pi // WID2) == (qi % GRP), W2e, 0.0)
    b2c = jnp.transpose(b2_ref[...])
    b2p = jnp.broadcast_to(b2c[None], (GRP, WID2, 1)).reshape(GRP * WID2, 1)

    W1c = W1_ref[...]
    b1c = b1_ref[...]

    t_refs = (t0_ref, t1_ref, t2_ref, t3_ref)
    r_refs = (r0_ref, r1_ref, r2s_ref, r3_ref)

    def body(g, carry):
        snc_g = snc_ref[pl.ds(g * GRP, GRP), :]
        Hg = jnp.tanh(W1c * snc_g[None] + b1c)
        Hg = Hg.reshape(WID1 * GRP, N)
        Gg = jnp.tanh(
            jax.lax.dot_general(w2p_ref[...], Hg, (((1,), (0,)), ((), ())),
                                preferred_element_type=jnp.float32) + b2p)
        Gg3 = Gg.reshape(GRP, WID2, N)
        for x in range(4):
            Rg = r_refs[x][pl.ds(g * GRP, GRP), :][:, None, :]
            t_refs[x][pl.ds(g * GRP, GRP), :] = (
                jnp.sum(Gg3 * Rg, axis=2) * (1.0 / NNBRS))
        return carry

    jax.lax.fori_loop(0, N // GRP, body, 0)

    Tm = [t0_ref[...], t1_ref[...], t2_ref[...], t3_ref[...]]

    TN = Tm[0] + tb_ref[...]
    feats = []
    for a in range(AXIS):
        fa = TN * TN[:, a:a + 1]
        for d in range(1, 4):
            fa = fa + Tm[d] * Tm[d][:, a:a + 1]
        feats.append(fa)
    feat = jnp.concatenate(feats, axis=1)

    f1 = jnp.tanh(jnp.dot(feat, Wf1_ref[...],
                          preferred_element_type=jnp.float32) + bf1_ref[...])
    f2 = jnp.tanh(jnp.dot(f1, Wf2_ref[...],
                          preferred_element_type=jnp.float32) + bf2_ref[...])
    v = jnp.dot(jnp.sum(f2, axis=0, keepdims=True), Wf3_ref[...],
                preferred_element_type=jnp.float32)

    out_ref[...] = (v + N * (bf3_ref[0] + EBIAS)) * OUT_NORM


def kernel(coord_N3, box_33, W_e1, b_e1, W_e2, b_e2, Tbias,
           W_f1, b_f1, W_f2, b_f2, W_f3, b_f3):
    W1c = W_e1.reshape(WID1, 1, 1)
    b1c = b_e1.reshape(WID1, 1, 1)
    b2r = b_e2.reshape(1, WID2)
    tb = Tbias.reshape(1, WID2)
    bf1 = b_f1.reshape(1, FIT)
    bf2 = b_f2.reshape(1, FIT)

    f32 = jnp.float32
    res = pl.pallas_call(
        _dp_kernel,
        grid=(1,),
        in_specs=[
            pl.BlockSpec(memory_space=pltpu.SMEM),
            pl.BlockSpec(memory_space=pltpu.SMEM),
            pl.BlockSpec((N, 3), lambda i: (0, 0)),
            pl.BlockSpec((WID1, 1, 1), lambda i: (0, 0, 0)),
            pl.BlockSpec((WID1, 1, 1), lambda i: (0, 0, 0)),
            pl.BlockSpec((WID1, WID2), lambda i: (0, 0)),
            pl.BlockSpec((1, WID2), lambda i: (0, 0)),
            pl.BlockSpec((1, WID2), lambda i: (0, 0)),
            pl.BlockSpec((AXIS * WID2, FIT), lambda i: (0, 0)),
            pl.BlockSpec((1, FIT), lambda i: (0, 0)),
            pl.BlockSpec((FIT, FIT), lambda i: (0, 0)),
            pl.BlockSpec((1, FIT), lambda i: (0, 0)),
            pl.BlockSpec((FIT, 1), lambda i: (0, 0)),
        ],
        out_specs=pl.BlockSpec((1, 1), lambda i: (0, 0)),
        out_shape=jax.ShapeDtypeStruct((1, 1), f32),
        scratch_shapes=[
            pltpu.VMEM((N, N), f32),
            pltpu.VMEM((N, N), f32),
            pltpu.VMEM((N, N), f32),
            pltpu.VMEM((N, N), f32),
            pltpu.VMEM((N, N), f32),
            pltpu.VMEM((GRP * WID2, WID1 * GRP), f32),
            pltpu.VMEM((N, WID2), f32),
            pltpu.VMEM((N, WID2), f32),
            pltpu.VMEM((N, WID2), f32),
            pltpu.VMEM((N, WID2), f32),
        ],
    )(box_33, b_f3, coord_N3, W1c, b1c, W_e2, b2r, tb,
      W_f1, bf1, W_f2, bf2, W_f3)

    return res[0, 0]


# speedup vs baseline: 1.2386x; 1.2386x over previous
import jax
import jax.numpy as jnp
from jax.experimental import pallas as pl
from jax.experimental.pallas import tpu as pltpu

N = 512
RCUT = 6.0
RCUT_SMTH = 0.5
SR_MEAN = 0.1
SR_STD = 0.3
NNBRS = 128.0
AXIS = 16
WID1 = 32
WID2 = 64
FIT = 128
OUT_NORM = 1.0
EBIAS = 0.0

BLK = 128
GRP = 8


def _dp_kernel(box_ref, bf3_ref, cb_ref, cfull_ref, W1_ref, b1_ref, W2_ref,
               b2_ref, tb_ref, Wf1_ref, bf1_ref, Wf2_ref, bf2_ref, Wf3_ref,
               out_ref, ct_ref, w2p_ref, b2p_ref):
    i = pl.program_id(0)

    @pl.when(i == 0)
    def _prep():
        ct_ref[...] = jnp.transpose(cfull_ref[...])
        W2T = jnp.transpose(W2_ref[...])
        A = jnp.broadcast_to(W2T[None], (GRP, WID2, WID1)).reshape(
            GRP * WID2, WID1)
        ci = jax.lax.broadcasted_iota(jnp.int32, (WID1, WID1 * GRP), 0)
        cj = jax.lax.broadcasted_iota(jnp.int32, (WID1, WID1 * GRP), 1)
        Sel = jnp.where(ci == cj // GRP, 1.0, 0.0).astype(jnp.float32)
        W2e = jnp.dot(A, Sel, preferred_element_type=jnp.float32)
        pi = jax.lax.broadcasted_iota(jnp.int32, (GRP * WID2, WID1 * GRP), 0)
        qi = jax.lax.broadcasted_iota(jnp.int32, (GRP * WID2, WID1 * GRP), 1)
        w2p_ref[...] = jnp.where((pi // WID2) == (qi % GRP), W2e, 0.0)
        b2c = jnp.transpose(b2_ref[...])
        b2p_ref[...] = jnp.broadcast_to(b2c[None], (GRP, WID2, 1)).reshape(
            GRP * WID2, 1)

    cb = cb_ref[...]
    cT = ct_ref[...]

    diffs = []
    r2 = jnp.full((BLK, N), 1e-16, jnp.float32)
    for d in range(3):
        Ld = box_ref[d, d]
        dd = cb[:, d:d + 1] - cT[d:d + 1, :]
        fr = dd * (1.0 / Ld)
        fr = fr - jnp.round(fr)
        dd = fr * Ld
        diffs.append(dd)
        r2 = r2 + dd * dd
    r = jnp.sqrt(r2)

    u = (r - RCUT_SMTH) / (RCUT - RCUT_SMTH)
    u = jnp.clip(u, 0.0, 1.0)
    sw = u * u * u * (-6.0 * u * u + 15.0 * u - 10.0) + 1.0
    inv_r = 1.0 / jnp.maximum(r, 1e-8)
    sr = jnp.where(r < RCUT, inv_r * sw, 0.0)
    rows = i * BLK + jax.lax.broadcasted_iota(jnp.int32, (BLK, N), 0)
    cols = jax.lax.broadcasted_iota(jnp.int32, (BLK, N), 1)
    sr = jnp.where(rows == cols, 0.0, sr)

    srn = sr / SR_STD
    snc = (sr - SR_MEAN) / SR_STD
    inv_rr = 1.0 / (r + 1e-16)
    sq3 = 3.0 ** 0.5
    R = [srn] + [sq3 * srn * (diffs[d] * inv_rr) for d in range(3)]

    W1c = W1_ref[...]
    b1c = b1_ref[...]
    W2p = w2p_ref[...]
    b2p = b2p_ref[...]

    T = [[] for _ in range(4)]
    for g in range(BLK // GRP):
        snc_g = snc[g * GRP:(g + 1) * GRP, :]
        Hg = jnp.tanh(W1c * snc_g[None] + b1c)
        Hg = Hg.reshape(WID1 * GRP, N)
        Gg = jnp.tanh(
            jax.lax.dot_general(W2p, Hg, (((1,), (0,)), ((), ())),
                                preferred_element_type=jnp.float32) + b2p)
        Gg3 = Gg.reshape(GRP, WID2, N)
        for x in range(4):
            Rg = R[x][g * GRP:(g + 1) * GRP, None, :]
            T[x].append(jnp.sum(Gg3 * Rg, axis=2) * (1.0 / NNBRS))
    Tm = [jnp.concatenate(T[x], axis=0) for x in range(4)]

    TN = Tm[0] + tb_ref[...]
    feats = []
    for a in range(AXIS):
        fa = TN * TN[:, a:a + 1]
        for d in range(1, 4):
            fa = fa + Tm[d] * Tm[d][:, a:a + 1]
        feats.append(fa)
    feat = jnp.concatenate(feats, axis=1)

    f1 = jnp.tanh(jnp.dot(feat, Wf1_ref[...],
                          preferred_element_type=jnp.float32) + bf1_ref[...])
    f2 = jnp.tanh(jnp.dot(f1, Wf2_ref[...],
                          preferred_element_type=jnp.float32) + bf2_ref[...])
    v = jnp.dot(jnp.sum(f2, axis=0, keepdims=True), Wf3_ref[...],
                preferred_element_type=jnp.float32)

    @pl.when(i == 0)
    def _first():
        out_ref[...] = v

    @pl.when(i > 0)
    def _rest():
        out_ref[...] += v

    @pl.when(i == N // BLK - 1)
    def _final():
        out_ref[...] = (out_ref[...] + N * (bf3_ref[0] + EBIAS)) * OUT_NORM


def kernel(coord_N3, box_33, W_e1, b_e1, W_e2, b_e2, Tbias,
           W_f1, b_f1, W_f2, b_f2, W_f3, b_f3):
    W1c = W_e1.reshape(WID1, 1, 1)
    b1c = b_e1.reshape(WID1, 1, 1)
    b2r = b_e2.reshape(1, WID2)
    tb = Tbias.reshape(1, WID2)
    bf1 = b_f1.reshape(1, FIT)
    bf2 = b_f2.reshape(1, FIT)

    f32 = jnp.float32
    res = pl.pallas_call(
        _dp_kernel,
        grid=(N // BLK,),
        in_specs=[
            pl.BlockSpec(memory_space=pltpu.SMEM),
            pl.BlockSpec(memory_space=pltpu.SMEM),
            pl.BlockSpec((BLK, 3), lambda i: (i, 0)),
            pl.BlockSpec((N, 3), lambda i: (0, 0)),
            pl.BlockSpec((WID1, 1, 1), lambda i: (0, 0, 0)),
            pl.BlockSpec((WID1, 1, 1), lambda i: (0, 0, 0)),
            pl.BlockSpec((WID1, WID2), lambda i: (0, 0)),
            pl.BlockSpec((1, WID2), lambda i: (0, 0)),
            pl.BlockSpec((1, WID2), lambda i: (0, 0)),
            pl.BlockSpec((AXIS * WID2, FIT), lambda i: (0, 0)),
            pl.BlockSpec((1, FIT), lambda i: (0, 0)),
            pl.BlockSpec((FIT, FIT), lambda i: (0, 0)),
            pl.BlockSpec((1, FIT), lambda i: (0, 0)),
            pl.BlockSpec((FIT, 1), lambda i: (0, 0)),
        ],
        out_specs=pl.BlockSpec((1, 1), lambda i: (0, 0)),
        out_shape=jax.ShapeDtypeStruct((1, 1), f32),
        scratch_shapes=[
            pltpu.VMEM((3, N), f32),
            pltpu.VMEM((GRP * WID2, WID1 * GRP), f32),
            pltpu.VMEM((GRP * WID2, 1), f32),
        ],
    )(box_33, b_f3, coord_N3, coord_N3, W1c, b1c, W_e2, b2r, tb,
      W_f1, bf1, W_f2, bf2, W_f3)

    return res[0, 0]


# speedup vs baseline: 1.2990x; 1.0488x over previous
import jax
import jax.numpy as jnp
from jax.experimental import pallas as pl
from jax.experimental.pallas import tpu as pltpu

N = 512
RCUT = 6.0
RCUT_SMTH = 0.5
SR_MEAN = 0.1
SR_STD = 0.3
NNBRS = 128.0
AXIS = 16
WID1 = 32
WID2 = 64
FIT = 128
OUT_NORM = 1.0
EBIAS = 0.0

BLK = 256
GRP = 8


def _dp_kernel(box_ref, bf3_ref, cb_ref, cfull_ref, W1_ref, b1_ref, W2_ref,
               b2_ref, tb_ref, Wf1_ref, bf1_ref, Wf2_ref, bf2_ref, Wf3_ref,
               out_ref, ct_ref, w2p_ref, b2p_ref):
    i = pl.program_id(0)

    @pl.when(i == 0)
    def _prep():
        ct_ref[...] = jnp.transpose(cfull_ref[...])
        W2T = jnp.transpose(W2_ref[...])
        A = jnp.broadcast_to(W2T[None], (GRP, WID2, WID1)).reshape(
            GRP * WID2, WID1)
        ci = jax.lax.broadcasted_iota(jnp.int32, (WID1, WID1 * GRP), 0)
        cj = jax.lax.broadcasted_iota(jnp.int32, (WID1, WID1 * GRP), 1)
        Sel = jnp.where(ci == cj // GRP, 1.0, 0.0).astype(jnp.float32)
        W2e = jnp.dot(A, Sel, preferred_element_type=jnp.float32)
        pi = jax.lax.broadcasted_iota(jnp.int32, (GRP * WID2, WID1 * GRP), 0)
        qi = jax.lax.broadcasted_iota(jnp.int32, (GRP * WID2, WID1 * GRP), 1)
        w2p_ref[...] = jnp.where((pi // WID2) == (qi % GRP), W2e, 0.0)
        b2c = jnp.transpose(b2_ref[...])
        b2p_ref[...] = jnp.broadcast_to(b2c[None], (GRP, WID2, 1)).reshape(
            GRP * WID2, 1)

    cb = cb_ref[...]
    cT = ct_ref[...]

    diffs = []
    r2 = jnp.full((BLK, N), 1e-16, jnp.float32)
    for d in range(3):
        Ld = box_ref[d, d]
        dd = cb[:, d:d + 1] - cT[d:d + 1, :]
        fr = dd * (1.0 / Ld)
        fr = fr - jnp.round(fr)
        dd = fr * Ld
        diffs.append(dd)
        r2 = r2 + dd * dd
    r = jnp.sqrt(r2)

    u = (r - RCUT_SMTH) / (RCUT - RCUT_SMTH)
    u = jnp.clip(u, 0.0, 1.0)
    sw = u * u * u * (-6.0 * u * u + 15.0 * u - 10.0) + 1.0
    inv_r = 1.0 / jnp.maximum(r, 1e-8)
    sr = jnp.where(r < RCUT, inv_r * sw, 0.0)
    rows = i * BLK + jax.lax.broadcasted_iota(jnp.int32, (BLK, N), 0)
    cols = jax.lax.broadcasted_iota(jnp.int32, (BLK, N), 1)
    sr = jnp.where(rows == cols, 0.0, sr)

    srn = sr / SR_STD
    snc = (sr - SR_MEAN) / SR_STD
    inv_rr = 1.0 / (r + 1e-16)
    sq3 = 3.0 ** 0.5
    R = [srn] + [sq3 * srn * (diffs[d] * inv_rr) for d in range(3)]

    W1c = W1_ref[...]
    b1c = b1_ref[...]
    W2p = w2p_ref[...]
    b2p = b2p_ref[...]

    T = [[] for _ in range(4)]
    for g in range(BLK // GRP):
        snc_g = snc[g * GRP:(g + 1) * GRP, :]
        Hg = jnp.tanh(W1c * snc_g[None] + b1c)
        Hg = Hg.reshape(WID1 * GRP, N)
        Gg = jnp.tanh(
            jax.lax.dot_general(W2p, Hg, (((1,), (0,)), ((), ())),
                                preferred_element_type=jnp.float32) + b2p)
        Gg3 = Gg.reshape(GRP, WID2, N)
        for x in range(4):
            Rg = R[x][g * GRP:(g + 1) * GRP, None, :]
            T[x].append(jnp.sum(Gg3 * Rg, axis=2) * (1.0 / NNBRS))
    Tm = [jnp.concatenate(T[x], axis=0) for x in range(4)]

    TN = Tm[0] + tb_ref[...]
    feats = []
    for a in range(AXIS):
        fa = TN * TN[:, a:a + 1]
        for d in range(1, 4):
            fa = fa + Tm[d] * Tm[d][:, a:a + 1]
        feats.append(fa)
    feat = jnp.concatenate(feats, axis=1)

    f1 = jnp.tanh(jnp.dot(feat, Wf1_ref[...],
                          preferred_element_type=jnp.float32) + bf1_ref[...])
    f2 = jnp.tanh(jnp.dot(f1, Wf2_ref[...],
                          preferred_element_type=jnp.float32) + bf2_ref[...])
    v = jnp.dot(jnp.sum(f2, axis=0, keepdims=True), Wf3_ref[...],
                preferred_element_type=jnp.float32)

    @pl.when(i == 0)
    def _first():
        out_ref[...] = v

    @pl.when(i > 0)
    def _rest():
        out_ref[...] += v

    @pl.when(i == N // BLK - 1)
    def _final():
        out_ref[...] = (out_ref[...] + N * (bf3_ref[0] + EBIAS)) * OUT_NORM


def kernel(coord_N3, box_33, W_e1, b_e1, W_e2, b_e2, Tbias,
           W_f1, b_f1, W_f2, b_f2, W_f3, b_f3):
    W1c = W_e1.reshape(WID1, 1, 1)
    b1c = b_e1.reshape(WID1, 1, 1)
    b2r = b_e2.reshape(1, WID2)
    tb = Tbias.reshape(1, WID2)
    bf1 = b_f1.reshape(1, FIT)
    bf2 = b_f2.reshape(1, FIT)

    f32 = jnp.float32
    res = pl.pallas_call(
        _dp_kernel,
        grid=(N // BLK,),
        in_specs=[
            pl.BlockSpec(memory_space=pltpu.SMEM),
            pl.BlockSpec(memory_space=pltpu.SMEM),
            pl.BlockSpec((BLK, 3), lambda i: (i, 0)),
            pl.BlockSpec((N, 3), lambda i: (0, 0)),
            pl.BlockSpec((WID1, 1, 1), lambda i: (0, 0, 0)),
            pl.BlockSpec((WID1, 1, 1), lambda i: (0, 0, 0)),
            pl.BlockSpec((WID1, WID2), lambda i: (0, 0)),
            pl.BlockSpec((1, WID2), lambda i: (0, 0)),
            pl.BlockSpec((1, WID2), lambda i: (0, 0)),
            pl.BlockSpec((AXIS * WID2, FIT), lambda i: (0, 0)),
            pl.BlockSpec((1, FIT), lambda i: (0, 0)),
            pl.BlockSpec((FIT, FIT), lambda i: (0, 0)),
            pl.BlockSpec((1, FIT), lambda i: (0, 0)),
            pl.BlockSpec((FIT, 1), lambda i: (0, 0)),
        ],
        out_specs=pl.BlockSpec((1, 1), lambda i: (0, 0)),
        out_shape=jax.ShapeDtypeStruct((1, 1), f32),
        scratch_shapes=[
            pltpu.VMEM((3, N), f32),
            pltpu.VMEM((GRP * WID2, WID1 * GRP), f32),
            pltpu.VMEM((GRP * WID2, 1), f32),
        ],
    )(box_33, b_f3, coord_N3, coord_N3, W1c, b1c, W_e2, b2r, tb,
      W_f1, bf1, W_f2, bf2, W_f3)

    return res[0, 0]
